# Initial kernel scaffold; baseline (speedup 1.0000x reference)
import jax
import jax.numpy as jnp
from jax.experimental import pallas as pl


def kernel(x, edge_index, edge_attr, u, batch, W1, b1, g1, bt1, W2, b2, W3, b3, g2, bt2, W4, b4):
    raise NotImplementedError("write your pallas kernel here")


# speedup vs baseline: 3.8528x; 3.8528x over previous
import functools

import jax
import jax.numpy as jnp
from jax import lax
from jax.experimental import pallas as pl
from jax.experimental.pallas import tpu as pltpu
from jax.experimental.pallas import tpu_sc as plsc

N = 10000
E = 320000
F = 128
H = 16

NC = 2
NS = 16
NW = NC * NS
TE = E // NW
CB = 125
NSUB = TE // CB
BIG = 2000
NBIG = TE // BIG
SPB = BIG // CB
NP = 10240
ROWU = 10

_HIGH = lax.Precision.HIGHEST


def _dot(a, b):
    return jnp.dot(a, b, precision=_HIGH, preferred_element_type=jnp.float32)


def _pq_body(x_ref, w1a_ref, w3a_ref, p_ref, q_ref):
    xv = x_ref[...]
    p_ref[...] = _dot(xv, w1a_ref[...])
    q_ref[...] = _dot(xv, w3a_ref[...])


def _edge_lin_body(ea_ref, w_ref, b_ref, o_ref):
    o_ref[...] = _dot(ea_ref[...], w_ref[...]) + b_ref[...]


def _stats_body(ps_ref, pq_ref, g1_ref, bt1_ref, o_ref):
    s1 = jnp.sum(ps_ref[...], axis=0, keepdims=True)
    s2 = jnp.sum(pq_ref[...], axis=0, keepdims=True)
    m = s1 / float(E)
    v = s2 / float(E) - m * m
    a = g1_ref[...] * lax.rsqrt(v + 1e-5)
    c = bt1_ref[...] - m * a
    o_ref[...] = jnp.concatenate([a, c], axis=0)


def _final_body(q_ref, a0_ref, a1_ref, w2_ref, b2_ref, w3b_ref, b3_ref,
                g2_ref, bt2_ref, w4_ref, b4_ref, o_ref):
    s = a0_ref[:, 0:H] + a1_ref[:, 0:H]
    cnt = a0_ref[:, H:H + 1] + a1_ref[:, H:H + 1]
    sm = s / jnp.maximum(cnt, 1.0)
    agg = _dot(sm, w2_ref[...]) + b2_ref[...] * (cnt > 0).astype(jnp.float32)
    z1 = q_ref[...] + _dot(agg, w3b_ref[...]) + b3_ref[...]
    m = jnp.mean(z1, axis=0, keepdims=True)
    v = jnp.mean(z1 * z1, axis=0, keepdims=True) - m * m
    zn = jnp.maximum((z1 - m) * lax.rsqrt(v + 1e-5) * g2_ref[...]
                     + bt2_ref[...], 0.0)
    o_ref[...] = _dot(zn, w4_ref[...]) + b4_ref[...]


_MESH = plsc.VectorSubcoreMesh(core_axis_name="c", subcore_axis_name="s")


@functools.partial(
    pl.kernel,
    mesh=_MESH,
    compiler_params=pltpu.CompilerParams(use_tc_tiling_on_sc=False),
    out_type=[
        jax.ShapeDtypeStruct((E, H), jnp.float32),
        jax.ShapeDtypeStruct((NW * 8, H), jnp.float32),
        jax.ShapeDtypeStruct((NW * 8, H), jnp.float32),
    ],
    scratch_types=[
        pltpu.VMEM((NSUB, CB), jnp.int32),
        pltpu.VMEM((BIG, H), jnp.float32),
        pltpu.VMEM((BIG, H), jnp.float32),
        pltpu.VMEM((8, H), jnp.float32),
        pltpu.VMEM((8, H), jnp.float32),
        pltpu.SemaphoreType.DMA,
        pltpu.SemaphoreType.DMA,
    ],
)
def _sc_pass1(p_hbm, a_hbm, send_hbm, h_hbm, psum_hbm, psumsq_hbm,
              idx_v, gbuf, abuf, s1v, s2v, gsem, dsem):
    cid = lax.axis_index("c")
    sid = lax.axis_index("s")
    wid = sid * NC + cid
    base = wid * TE

    pltpu.sync_copy(send_hbm.at[wid], idx_v)

    def big_body(b, carry):
        s1, s2 = carry
        row0 = base + b * BIG
        a_cp = pltpu.async_copy(a_hbm.at[pl.ds(row0, BIG)], abuf, dsem)
        gathers = []
        for j in range(SPB):
            gathers.append(pltpu.async_copy(
                p_hbm.at[idx_v.at[b * SPB + j]],
                gbuf.at[pl.ds(j * CB, CB)], gsem))
        a_cp.wait()
        for g in gathers:
            g.wait()

        def row_body(r0, carry2):
            t1, t2 = carry2
            for u in range(ROWU):
                r = r0 * ROWU + u
                hv = gbuf[r] + abuf[r]
                gbuf[r] = hv
                t1 = t1 + hv
                t2 = t2 + hv * hv
            return t1, t2

        s1, s2 = lax.fori_loop(0, BIG // ROWU, row_body, (s1, s2),
                               unroll=False)
        pltpu.sync_copy(gbuf, h_hbm.at[pl.ds(row0, BIG)])
        return s1, s2

    zero = jnp.zeros((H,), jnp.float32)
    for r in range(8):
        s1v[r] = zero
        s2v[r] = zero
    s1, s2 = lax.fori_loop(0, NBIG, big_body, (zero, zero), unroll=False)
    s1v[0] = s1
    s2v[0] = s2
    pltpu.sync_copy(s1v, psum_hbm.at[pl.ds(wid * 8, 8)])
    pltpu.sync_copy(s2v, psumsq_hbm.at[pl.ds(wid * 8, 8)])


@functools.partial(
    pl.kernel,
    mesh=_MESH,
    compiler_params=pltpu.CompilerParams(use_tc_tiling_on_sc=False),
    out_type=jax.ShapeDtypeStruct((NC * NP, 2 * H), jnp.float32),
    scratch_types=[
        pltpu.VMEM((NSUB, CB), jnp.int32),
        pltpu.VMEM((BIG, H), jnp.float32),
        pltpu.VMEM((BIG, 2 * H), jnp.float32),
        pltpu.VMEM((2, H), jnp.float32),
        pltpu.VMEM_SHARED((NP, 2 * H), jnp.float32),
        pltpu.SemaphoreType.DMA,
    ],
)
def _sc_pass2(h_hbm, rec_hbm, ac_hbm, const_hbm, zeros_hbm, out_hbm,
              idx_v, hbuf, sbuf, acv, acc, dsem):
    cid = lax.axis_index("c")
    sid = lax.axis_index("s")
    wid = sid * NC + cid
    base = wid * TE

    @pl.when(sid == 0)
    def _():
        pltpu.sync_copy(zeros_hbm, acc)

    pltpu.sync_copy(const_hbm, sbuf)
    pltpu.sync_copy(ac_hbm, acv)
    pltpu.sync_copy(rec_hbm.at[wid], idx_v)
    plsc.subcore_barrier()

    av = acv[0]
    cv = acv[1]

    def big_body(b, carry):
        row0 = base + b * BIG
        pltpu.async_copy(h_hbm.at[pl.ds(row0, BIG)], hbuf, dsem).wait()

        def row_body(r0, carry2):
            for u in range(ROWU):
                r = r0 * ROWU + u
                sbuf[r, 0:H] = jnp.maximum(hbuf[r] * av + cv, 0.0)
            return carry2

        lax.fori_loop(0, BIG // ROWU, row_body, 0, unroll=False)
        for j in range(SPB):
            pltpu.sync_copy(sbuf.at[pl.ds(j * CB, CB)],
                            acc.at[idx_v.at[b * SPB + j]], add=True)
        return carry

    lax.fori_loop(0, NBIG, big_body, 0, unroll=False)
    plsc.subcore_barrier()
    rows = NP // NS
    pltpu.sync_copy(acc.at[pl.ds(sid * rows, rows)],
                    out_hbm.at[pl.ds(cid * NP + sid * rows, rows)])


def kernel(x, edge_index, edge_attr, u, batch, W1, b1, g1, bt1, W2, b2,
           W3, b3, g2, bt2, W4, b4):
    del u, batch
    send = edge_index[0].astype(jnp.int32).reshape(NW, NSUB, CB)
    rec = edge_index[1].astype(jnp.int32).reshape(NW, NSUB, CB)

    p, q = pl.pallas_call(
        _pq_body,
        out_shape=[jax.ShapeDtypeStruct((N, H), jnp.float32),
                   jax.ShapeDtypeStruct((N, H), jnp.float32)],
    )(x, W1[:F], W3[:F])

    eblk = 10000
    a_mat = pl.pallas_call(
        _edge_lin_body,
        grid=(E // eblk,),
        in_specs=[pl.BlockSpec((eblk, H), lambda i: (i, 0)),
                  pl.BlockSpec((H, H), lambda i: (0, 0)),
                  pl.BlockSpec((1, H), lambda i: (0, 0))],
        out_specs=pl.BlockSpec((eblk, H), lambda i: (i, 0)),
        out_shape=jax.ShapeDtypeStruct((E, H), jnp.float32),
    )(edge_attr, W1[F:], b1[None])

    h, psum, psumsq = _sc_pass1(p, a_mat, send)

    ac = pl.pallas_call(
        _stats_body,
        out_shape=jax.ShapeDtypeStruct((2, H), jnp.float32),
    )(psum, psumsq, g1[None], bt1[None])

    const = jnp.zeros((BIG, 2 * H), jnp.float32).at[:, H].set(1.0)
    zeros = jnp.zeros((NP, 2 * H), jnp.float32)
    acc = _sc_pass2(h, rec, ac, const, zeros)
    acc = acc.reshape(NC, NP, 2 * H)[:, :N, :]

    z = pl.pallas_call(
        _final_body,
        out_shape=jax.ShapeDtypeStruct((N, H), jnp.float32),
    )(q, acc[0], acc[1], W2, b2[None], W3[F:], b3[None],
      g2[None], bt2[None], W4, b4[None])
    return z
